# baseline (speedup 1.0000x reference)
import functools

import jax
import jax.numpy as jnp
from jax import lax
from jax.experimental import pallas as pl
from jax.experimental.pallas import tpu as pltpu
from jax.experimental.pallas import tpu_sc as plsc

N = 10000
D = 128
DA = 144
NPAD = 10240
E = 320000
NW = 32
EW = 10240
EP = NW * EW
C = 64
CH = EW // C
NBUF = 2
RPT = NPAD // 16


def _sc_aggregate(xa, spk, dpk):
    @functools.partial(
        pl.kernel,
        out_type=jax.ShapeDtypeStruct((2, NPAD, DA), jnp.float32),
        mesh=plsc.VectorSubcoreMesh(core_axis_name="c", subcore_axis_name="s"),
        compiler_params=pltpu.CompilerParams(
            needs_layout_passes=False, use_tc_tiling_on_sc=False),
        scratch_types=[
            pltpu.VMEM_SHARED((NPAD, DA), jnp.float32),
            pltpu.VMEM((CH, C // 2), jnp.int32),
            pltpu.VMEM((CH, C // 2), jnp.int32),
            pltpu.VMEM((NBUF, C), jnp.int32),
            pltpu.VMEM((NBUF, C), jnp.int32),
            pltpu.VMEM((NBUF, C, DA), jnp.float32),
            pltpu.SemaphoreType.DMA((NBUF,)),
            pltpu.SemaphoreType.DMA((NBUF,)),
        ],
    )
    def k(x_hbm, spk_hbm, dpk_hbm, part_hbm,
          acc_sh, spk_v, dpk_v, sidx, didx, rows, gsem, ssem):
        cid = lax.axis_index("c")
        sid = lax.axis_index("s")
        wid = (1 - cid) * 16 + sid
        zeros16 = jnp.zeros((16,), jnp.float32)

        with jax.named_scope("zero_stage"):
            pltpu.sync_copy(spk_hbm.at[wid], spk_v)
            pltpu.sync_copy(dpk_hbm.at[wid], dpk_v)

            def zrow(r, carry):
                for i in range(DA // 16):
                    rows[0, r, pl.ds(i * 16, 16)] = zeros16
                return carry

            lax.fori_loop(0, C, zrow, 0)
            for kk in range(RPT // C):
                pltpu.sync_copy(rows.at[0],
                                acc_sh.at[pl.ds(sid * RPT + kk * C, C)])
            plsc.subcore_barrier()

        def unpack(pk_ref, out_ref, b, g):
            for grp in range(C // 32):
                w = pk_ref[g, pl.ds(grp * 16, 16)]
                out_ref[b, pl.ds(grp * 32, 16)] = w & 0xFFFF
                out_ref[b, pl.ds(grp * 32 + 16, 16)] = lax.shift_right_logical(w, 16)

        for b in range(NBUF):
            unpack(spk_v, sidx, b, b)
            unpack(dpk_v, didx, b, b)
            pltpu.async_copy(x_hbm.at[sidx.at[b]], rows.at[b], gsem.at[b])

        def outer(o, carry):
            for b in range(NBUF):
                g = o * NBUF + b
                pltpu.make_async_copy(
                    x_hbm.at[sidx.at[b]], rows.at[b], gsem.at[b]).wait()
                pltpu.async_copy(
                    rows.at[b], acc_sh.at[didx.at[b]], ssem.at[b],
                    add=True).wait()

                @pl.when(g < CH - NBUF)
                def _():
                    unpack(spk_v, sidx, b, g + NBUF)
                    unpack(dpk_v, didx, b, g + NBUF)
                    pltpu.async_copy(
                        x_hbm.at[sidx.at[b]], rows.at[b], gsem.at[b])
            return carry

        with jax.named_scope("main_loop"):
            lax.fori_loop(0, CH // NBUF, outer, 0)
            plsc.subcore_barrier()

        with jax.named_scope("writeout"):
            for kk in range(RPT // C):
                r0 = sid * RPT + kk * C
                pltpu.sync_copy(acc_sh.at[pl.ds(r0, C)],
                                part_hbm.at[cid, pl.ds(r0, C)])

    return k(xa, spk, dpk)


def _tc_layer(part, xp, w1lT, b1r, w1rT, w2):
    def body(part_ref, x_ref, wl_ref, b1_ref, wr_ref, w2_ref,
             pq_ref, cntc_ref):
        sa = part_ref[0] + part_ref[1]
        s = sa[:, :D]
        cntc = jnp.maximum(sa[:, D], 1.0)
        aggr = s / cntc[:, None]
        h = jnp.maximum(
            jnp.dot(aggr, wl_ref[...], preferred_element_type=jnp.float32)
            + b1_ref[...]
            + jnp.dot(x_ref[...], wr_ref[...], preferred_element_type=jnp.float32),
            0.0)
        pq_ref[...] = jnp.dot(h, w2_ref[...], preferred_element_type=jnp.float32)
        cntc_ref[...] = cntc

    B = 512
    grid = NPAD // B
    return pl.pallas_call(
        body,
        grid=(grid,),
        in_specs=[
            pl.BlockSpec((2, B, DA), lambda i: (0, i, 0)),
            pl.BlockSpec((B, D), lambda i: (i, 0)),
            pl.BlockSpec((D, D), lambda i: (0, 0)),
            pl.BlockSpec((1, D), lambda i: (0, 0)),
            pl.BlockSpec((D, D), lambda i: (0, 0)),
            pl.BlockSpec((D, 2), lambda i: (0, 0)),
        ],
        out_specs=[
            pl.BlockSpec((B, 2), lambda i: (i, 0)),
            pl.BlockSpec((B,), lambda i: (i,)),
        ],
        out_shape=[
            jax.ShapeDtypeStruct((NPAD, 2), jnp.float32),
            jax.ShapeDtypeStruct((NPAD,), jnp.float32),
        ],
    )(part, xp, w1lT, b1r, w1rT, w2)


def _sc_scalar_aggregate(p, srcp, dstp):
    @functools.partial(
        pl.kernel,
        out_type=jax.ShapeDtypeStruct((NW, NPAD), jnp.float32),
        mesh=plsc.VectorSubcoreMesh(core_axis_name="c", subcore_axis_name="s"),
        compiler_params=pltpu.CompilerParams(needs_layout_passes=False),
        scratch_types=[
            pltpu.VMEM((NPAD,), jnp.float32),
            pltpu.VMEM((NPAD,), jnp.float32),
            pltpu.VMEM((EW,), jnp.int32),
            pltpu.VMEM((EW,), jnp.int32),
        ],
    )
    def k(p_hbm, src_hbm, dst_hbm, out_hbm, p_v, acc_v, sidx_v, didx_v):
        cid = lax.axis_index("c")
        sid = lax.axis_index("s")
        wid = cid * 16 + sid
        pltpu.sync_copy(p_hbm, p_v)
        pltpu.sync_copy(src_hbm.at[pl.ds(wid * EW, EW)], sidx_v)
        pltpu.sync_copy(dst_hbm.at[pl.ds(wid * EW, EW)], didx_v)
        zeros16 = jnp.zeros((16,), jnp.float32)

        def zacc(j, carry):
            acc_v[pl.ds(j * 16, 16)] = zeros16
            return carry

        lax.fori_loop(0, NPAD // 16, zacc, 0)

        def step(j, carry):
            si = sidx_v[pl.ds(j * 16, 16)]
            dv = didx_v[pl.ds(j * 16, 16)]
            vals = plsc.load_gather(p_v, [si])
            plsc.addupdate_scatter(acc_v, [dv], vals)
            return carry

        lax.fori_loop(0, EW // 16, step, 0)
        pltpu.sync_copy(acc_v, out_hbm.at[wid])

    return k(p, srcp, dstp)


def _tc_final(pacc_r, cntc_r, q_r, noise_r, scal):
    def body(sc_ref, pacc_ref, cntc_ref, q_ref, noise_ref, out_ref):
        a = jnp.sum(pacc_ref[...], axis=0)
        h2 = a / cntc_ref[...] + sc_ref[0] + q_ref[...]
        out_ref[...] = (h2 * sc_ref[1] + sc_ref[2]
                        + noise_ref[...] * jnp.exp(h2 * sc_ref[3] + sc_ref[4]))

    R = NPAD // 128
    return pl.pallas_call(
        body,
        in_specs=[
            pl.BlockSpec(memory_space=pltpu.SMEM),
            pl.BlockSpec((NW, R, 128), lambda: (0, 0, 0)),
            pl.BlockSpec((R, 128), lambda: (0, 0)),
            pl.BlockSpec((R, 128), lambda: (0, 0)),
            pl.BlockSpec((R, 128), lambda: (0, 0)),
        ],
        out_specs=pl.BlockSpec((R, 128), lambda: (0, 0)),
        out_shape=jax.ShapeDtypeStruct((R, 128), jnp.float32),
    )(scal, pacc_r, cntc_r, q_r, noise_r)


def kernel(x, edge_index, W1l, b1, W1r, W2l, b2, W2r, Wal, ba, War, wm, bm, wv, bv):
    src = edge_index[0].astype(jnp.int32)
    dst = edge_index[1].astype(jnp.int32)
    pad_e = EP - E
    srcp = jnp.concatenate([src, jnp.zeros((pad_e,), jnp.int32)])
    dummy = N + (jnp.arange(pad_e, dtype=jnp.int32) % (NPAD - N))
    dstp = jnp.concatenate([dst, dummy])
    xp = jnp.pad(x, ((0, NPAD - N), (0, 0)))
    xa = jnp.pad(x, ((0, NPAD - N), (0, DA - D)))
    xa = xa.at[:, D].set(1.0)

    def pack16(a):
        a5 = a.reshape(NW, CH, C // 32, 2, 16)
        return (a5[..., 0, :] | (a5[..., 1, :] << 16)).reshape(NW, CH, C // 2)

    part = _sc_aggregate(xa, pack16(srcp), pack16(dstp))

    w2 = jnp.stack([W2l[0], W2r[0]], axis=1)
    pq, cntc = _tc_layer(part, xp, W1l.T, b1.reshape(1, D), W1r.T, w2)

    p = pq[:, 0]
    pacc = _sc_scalar_aggregate(p, srcp, dstp)

    R = NPAD // 128
    noise = jax.random.normal(jax.random.key(42), (N, 1), jnp.float32)
    noise_r = jnp.pad(noise[:, 0], (0, NPAD - N)).reshape(R, 128)
    q_r = pq[:, 1].reshape(R, 128)
    cntc_r = cntc.reshape(R, 128)
    pacc_r = pacc.reshape(NW, R, 128)
    scal = jnp.concatenate([b2, wm.ravel(), bm, wv.ravel(), bv])

    out_r = _tc_final(pacc_r, cntc_r, q_r, noise_r, scal)
    return out_r.reshape(NPAD)[:N][:, None]


# speedup vs baseline: 2.6734x; 2.6734x over previous
import functools

import jax
import jax.numpy as jnp
from jax import lax
from jax.experimental import pallas as pl
from jax.experimental.pallas import tpu as pltpu
from jax.experimental.pallas import tpu_sc as plsc

N = 10000
D = 128
DA = 144
NPAD = 10240
E = 320000
NW = 32
EW = 10240
EP = NW * EW
C = 64
CH = EW // C
NBUF = 2
RPT = NPAD // 16


def _sc_aggregate(xa, spk, dpk):
    @functools.partial(
        pl.kernel,
        out_type=jax.ShapeDtypeStruct((2, NPAD, DA), jnp.float32),
        mesh=plsc.VectorSubcoreMesh(core_axis_name="c", subcore_axis_name="s"),
        compiler_params=pltpu.CompilerParams(
            needs_layout_passes=False, use_tc_tiling_on_sc=False),
        scratch_types=[
            pltpu.VMEM_SHARED((NPAD, DA), jnp.float32),
            pltpu.VMEM((CH, C // 2), jnp.int32),
            pltpu.VMEM((CH, C // 2), jnp.int32),
            pltpu.VMEM((NBUF, C), jnp.int32),
            pltpu.VMEM((NBUF, C), jnp.int32),
            pltpu.VMEM((NBUF, C, DA), jnp.float32),
            pltpu.SemaphoreType.DMA((NBUF,)),
            pltpu.SemaphoreType.DMA((NBUF,)),
        ],
    )
    def k(x_hbm, spk_hbm, dpk_hbm, part_hbm,
          acc_sh, spk_v, dpk_v, sidx, didx, rows, gsem, ssem):
        cid = lax.axis_index("c")
        sid = lax.axis_index("s")
        wid = cid * 16 + sid
        zeros16 = jnp.zeros((16,), jnp.float32)

        with jax.named_scope("zero_stage"):
            pltpu.sync_copy(spk_hbm.at[wid], spk_v)
            pltpu.sync_copy(dpk_hbm.at[wid], dpk_v)

            def zrow(r, carry):
                for i in range(DA // 16):
                    rows[0, r, pl.ds(i * 16, 16)] = zeros16
                return carry

            lax.fori_loop(0, C, zrow, 0)
            for kk in range(RPT // C):
                pltpu.sync_copy(rows.at[0],
                                acc_sh.at[pl.ds(sid * RPT + kk * C, C)])
            plsc.subcore_barrier()

        def unpack(pk_ref, out_ref, b, g):
            for grp in range(C // 32):
                w = pk_ref[g, pl.ds(grp * 16, 16)]
                out_ref[b, pl.ds(grp * 32, 16)] = w & 0xFFFF
                out_ref[b, pl.ds(grp * 32 + 16, 16)] = lax.shift_right_logical(w, 16)

        for b in range(NBUF):
            unpack(spk_v, sidx, b, b)
            unpack(dpk_v, didx, b, b)
            pltpu.async_copy(x_hbm.at[sidx.at[b]], rows.at[b], gsem.at[b])

        def outer(o, carry):
            for b in range(NBUF):
                g = o * NBUF + b
                pltpu.make_async_copy(
                    x_hbm.at[sidx.at[b]], rows.at[b], gsem.at[b]).wait()
                pltpu.async_copy(
                    rows.at[b], acc_sh.at[didx.at[b]], ssem.at[b],
                    add=True).wait()

                @pl.when(g < CH - NBUF)
                def _():
                    unpack(spk_v, sidx, b, g + NBUF)
                    unpack(dpk_v, didx, b, g + NBUF)
                    pltpu.async_copy(
                        x_hbm.at[sidx.at[b]], rows.at[b], gsem.at[b])
            return carry

        with jax.named_scope("main_loop"):
            lax.fori_loop(0, CH // NBUF, outer, 0)
            plsc.subcore_barrier()

        with jax.named_scope("writeout"):
            for kk in range(RPT // C):
                r0 = sid * RPT + kk * C
                pltpu.sync_copy(acc_sh.at[pl.ds(r0, C)],
                                part_hbm.at[cid, pl.ds(r0, C)])

    return k(xa, spk, dpk)


def _tc_layer(part, xp, w1lT, b1r, w1rT, w2):
    def body(part_ref, x_ref, wl_ref, b1_ref, wr_ref, w2_ref,
             pq_ref, cntc_ref):
        sa = part_ref[0] + part_ref[1]
        s = sa[:, :D]
        cntc = jnp.maximum(sa[:, D], 1.0)
        aggr = s / cntc[:, None]
        h = jnp.maximum(
            jnp.dot(aggr, wl_ref[...], preferred_element_type=jnp.float32)
            + b1_ref[...]
            + jnp.dot(x_ref[...], wr_ref[...], preferred_element_type=jnp.float32),
            0.0)
        pq_ref[...] = jnp.dot(h, w2_ref[...], preferred_element_type=jnp.float32)
        cntc_ref[...] = cntc

    B = 512
    grid = NPAD // B
    return pl.pallas_call(
        body,
        grid=(grid,),
        in_specs=[
            pl.BlockSpec((2, B, DA), lambda i: (0, i, 0)),
            pl.BlockSpec((B, D), lambda i: (i, 0)),
            pl.BlockSpec((D, D), lambda i: (0, 0)),
            pl.BlockSpec((1, D), lambda i: (0, 0)),
            pl.BlockSpec((D, D), lambda i: (0, 0)),
            pl.BlockSpec((D, 2), lambda i: (0, 0)),
        ],
        out_specs=[
            pl.BlockSpec((B, 2), lambda i: (i, 0)),
            pl.BlockSpec((B,), lambda i: (i,)),
        ],
        out_shape=[
            jax.ShapeDtypeStruct((NPAD, 2), jnp.float32),
            jax.ShapeDtypeStruct((NPAD,), jnp.float32),
        ],
    )(part, xp, w1lT, b1r, w1rT, w2)


def _sc_scalar_aggregate(p, srcp, dstp):
    @functools.partial(
        pl.kernel,
        out_type=jax.ShapeDtypeStruct((NW, NPAD), jnp.float32),
        mesh=plsc.VectorSubcoreMesh(core_axis_name="c", subcore_axis_name="s"),
        compiler_params=pltpu.CompilerParams(needs_layout_passes=False),
        scratch_types=[
            pltpu.VMEM((NPAD,), jnp.float32),
            pltpu.VMEM((NPAD,), jnp.float32),
            pltpu.VMEM((EW,), jnp.int32),
            pltpu.VMEM((EW,), jnp.int32),
        ],
    )
    def k(p_hbm, src_hbm, dst_hbm, out_hbm, p_v, acc_v, sidx_v, didx_v):
        cid = lax.axis_index("c")
        sid = lax.axis_index("s")
        wid = cid * 16 + sid
        pltpu.sync_copy(p_hbm, p_v)
        pltpu.sync_copy(src_hbm.at[pl.ds(wid * EW, EW)], sidx_v)
        pltpu.sync_copy(dst_hbm.at[pl.ds(wid * EW, EW)], didx_v)
        zeros16 = jnp.zeros((16,), jnp.float32)

        def zacc(j, carry):
            acc_v[pl.ds(j * 16, 16)] = zeros16
            return carry

        lax.fori_loop(0, NPAD // 16, zacc, 0)

        def step(j, carry):
            si = sidx_v[pl.ds(j * 16, 16)]
            dv = didx_v[pl.ds(j * 16, 16)]
            vals = plsc.load_gather(p_v, [si])
            plsc.addupdate_scatter(acc_v, [dv], vals)
            return carry

        lax.fori_loop(0, EW // 16, step, 0)
        pltpu.sync_copy(acc_v, out_hbm.at[wid])

    return k(p, srcp, dstp)


def _tc_final(pacc_r, cntc_r, q_r, noise_r, scal):
    def body(sc_ref, pacc_ref, cntc_ref, q_ref, noise_ref, out_ref):
        a = jnp.sum(pacc_ref[...], axis=0)
        h2 = a / cntc_ref[...] + sc_ref[0] + q_ref[...]
        out_ref[...] = (h2 * sc_ref[1] + sc_ref[2]
                        + noise_ref[...] * jnp.exp(h2 * sc_ref[3] + sc_ref[4]))

    R = NPAD // 128
    return pl.pallas_call(
        body,
        in_specs=[
            pl.BlockSpec(memory_space=pltpu.SMEM),
            pl.BlockSpec((NW, R, 128), lambda: (0, 0, 0)),
            pl.BlockSpec((R, 128), lambda: (0, 0)),
            pl.BlockSpec((R, 128), lambda: (0, 0)),
            pl.BlockSpec((R, 128), lambda: (0, 0)),
        ],
        out_specs=pl.BlockSpec((R, 128), lambda: (0, 0)),
        out_shape=jax.ShapeDtypeStruct((R, 128), jnp.float32),
    )(scal, pacc_r, cntc_r, q_r, noise_r)


def kernel(x, edge_index, W1l, b1, W1r, W2l, b2, W2r, Wal, ba, War, wm, bm, wv, bv):
    src = edge_index[0].astype(jnp.int32)
    dst = edge_index[1].astype(jnp.int32)
    pad_w = EW - E // NW
    pad_src = jnp.broadcast_to((jnp.arange(pad_w, dtype=jnp.int32) * 37) % N,
                               (NW, pad_w))
    pad_dst = jnp.broadcast_to(N + jnp.arange(pad_w, dtype=jnp.int32),
                               (NW, pad_w))
    srcp = jnp.concatenate([src.reshape(NW, E // NW), pad_src], axis=1).reshape(EP)
    dstp = jnp.concatenate([dst.reshape(NW, E // NW), pad_dst], axis=1).reshape(EP)
    xp = jnp.pad(x, ((0, NPAD - N), (0, 0)))
    xa = jnp.pad(x, ((0, NPAD - N), (0, DA - D)))
    xa = xa.at[:, D].set(1.0)

    def pack16(a):
        a5 = a.reshape(NW, CH, C // 32, 2, 16)
        return (a5[..., 0, :] | (a5[..., 1, :] << 16)).reshape(NW, CH, C // 2)

    part = _sc_aggregate(xa, pack16(srcp), pack16(dstp))

    w2 = jnp.stack([W2l[0], W2r[0]], axis=1)
    pq, cntc = _tc_layer(part, xp, W1l.T, b1.reshape(1, D), W1r.T, w2)

    p = pq[:, 0]
    pacc = _sc_scalar_aggregate(p, srcp, dstp)

    R = NPAD // 128
    noise = jax.random.normal(jax.random.key(42), (N, 1), jnp.float32)
    noise_r = jnp.pad(noise[:, 0], (0, NPAD - N)).reshape(R, 128)
    q_r = pq[:, 1].reshape(R, 128)
    cntc_r = cntc.reshape(R, 128)
    pacc_r = pacc.reshape(NW, R, 128)
    scal = jnp.concatenate([b2, wm.ravel(), bm, wv.ravel(), bv])

    out_r = _tc_final(pacc_r, cntc_r, q_r, noise_r, scal)
    return out_r.reshape(NPAD)[:N][:, None]


# speedup vs baseline: 2.7884x; 1.0430x over previous
import functools

import jax
import jax.numpy as jnp
from jax import lax
from jax.experimental import pallas as pl
from jax.experimental.pallas import tpu as pltpu
from jax.experimental.pallas import tpu_sc as plsc

N = 10000
D = 128
NPAD = 10240
E = 320000
NW = 32
EW = 10240
EP = NW * EW
C = 64
CH = EW // C
NBUF = 2
RPT = NPAD // 16


def _sc_aggregate(x, spk, dpk):
    @functools.partial(
        pl.kernel,
        out_type=[
            jax.ShapeDtypeStruct((2, NPAD, D), jnp.float32),
            jax.ShapeDtypeStruct((NW, NPAD), jnp.float32),
        ],
        mesh=plsc.VectorSubcoreMesh(core_axis_name="c", subcore_axis_name="s"),
        compiler_params=pltpu.CompilerParams(needs_layout_passes=False),
        scratch_types=[
            pltpu.VMEM_SHARED((NPAD, D), jnp.float32),
            pltpu.VMEM((EW // 2,), jnp.int32),
            pltpu.VMEM((EW // 2,), jnp.int32),
            pltpu.VMEM((NBUF, C), jnp.int32),
            pltpu.VMEM((NBUF, C), jnp.int32),
            pltpu.VMEM((NBUF, C, D), jnp.float32),
            pltpu.VMEM((NPAD,), jnp.float32),
            pltpu.SemaphoreType.DMA((NBUF,)),
            pltpu.SemaphoreType.DMA((NBUF,)),
        ],
    )
    def k(x_hbm, spk_hbm, dpk_hbm, part_hbm, cnt_hbm,
          acc_sh, spk_v, dpk_v, sidx, didx, rows, cnt_v, gsem, ssem):
        cid = lax.axis_index("c")
        sid = lax.axis_index("s")
        wid = cid * 16 + sid
        zeros16 = jnp.zeros((16,), jnp.float32)
        ones16 = jnp.ones((16,), jnp.float32)

        with jax.named_scope("zero_stage"):
            pltpu.sync_copy(spk_hbm.at[wid], spk_v)
            pltpu.sync_copy(dpk_hbm.at[wid], dpk_v)

            def zcnt(j, carry):
                cnt_v[pl.ds(j * 16, 16)] = zeros16
                return carry

            lax.fori_loop(0, NPAD // 16, zcnt, 0)

            def zrow(r, carry):
                for i in range(D // 16):
                    rows[0, r, pl.ds(i * 16, 16)] = zeros16
                return carry

            lax.fori_loop(0, C, zrow, 0)
            for kk in range(RPT // C):
                pltpu.sync_copy(rows.at[0],
                                acc_sh.at[pl.ds(sid * RPT + kk * C, C)])
            plsc.subcore_barrier()

        def unpack(pk_ref, out_ref, b, g):
            for grp in range(C // 32):
                w = pk_ref[pl.ds(g * (C // 2) + grp * 16, 16)]
                out_ref[b, pl.ds(grp * 32, 16)] = w & 0xFFFF
                out_ref[b, pl.ds(grp * 32 + 16, 16)] = lax.shift_right_logical(w, 16)

        for b in range(NBUF):
            unpack(spk_v, sidx, b, b)
            unpack(dpk_v, didx, b, b)
            pltpu.async_copy(x_hbm.at[sidx.at[b]], rows.at[b], gsem.at[b])

        def outer(o, carry):
            for b in range(NBUF):
                g = o * NBUF + b
                pltpu.make_async_copy(
                    x_hbm.at[sidx.at[b]], rows.at[b], gsem.at[b]).wait()
                sc_d = pltpu.async_copy(
                    rows.at[b], acc_sh.at[didx.at[b]], ssem.at[b], add=True)
                for t in range(C // 16):
                    dv = didx[b, pl.ds(t * 16, 16)]
                    plsc.addupdate_scatter(cnt_v, [dv], ones16)
                sc_d.wait()

                @pl.when(g < CH - NBUF)
                def _():
                    unpack(spk_v, sidx, b, g + NBUF)
                    unpack(dpk_v, didx, b, g + NBUF)
                    pltpu.async_copy(
                        x_hbm.at[sidx.at[b]], rows.at[b], gsem.at[b])
            return carry

        with jax.named_scope("main_loop"):
            lax.fori_loop(0, CH // NBUF, outer, 0)
            plsc.subcore_barrier()

        with jax.named_scope("writeout"):
            for kk in range(RPT // C):
                r0 = sid * RPT + kk * C
                pltpu.sync_copy(acc_sh.at[pl.ds(r0, C)],
                                part_hbm.at[cid, pl.ds(r0, C)])
            pltpu.sync_copy(cnt_v, cnt_hbm.at[wid])

    return k(x, spk, dpk)


def _tc_layer(part, cntp, xp, w1lT, b1r, w1rT, w2):
    def body(part_ref, cnt_ref, x_ref, wl_ref, b1_ref, wr_ref, w2_ref,
             p_ref, q_ref, cntc_ref):
        s = part_ref[0] + part_ref[1]
        cntc = jnp.maximum(jnp.sum(cnt_ref[...], axis=0), 1.0)
        aggr = s / cntc[:, None]
        h = jnp.maximum(
            jnp.dot(aggr, wl_ref[...], preferred_element_type=jnp.float32)
            + b1_ref[...]
            + jnp.dot(x_ref[...], wr_ref[...], preferred_element_type=jnp.float32),
            0.0)
        pq = jnp.dot(h, w2_ref[...], preferred_element_type=jnp.float32)
        p_ref[...] = pq[:, 0:1]
        q_ref[...] = pq[:, 1:2]
        cntc_ref[...] = cntc

    B = 512
    grid = NPAD // B
    return pl.pallas_call(
        body,
        grid=(grid,),
        in_specs=[
            pl.BlockSpec((2, B, D), lambda i: (0, i, 0)),
            pl.BlockSpec((NW, B), lambda i: (0, i)),
            pl.BlockSpec((B, D), lambda i: (i, 0)),
            pl.BlockSpec((D, D), lambda i: (0, 0)),
            pl.BlockSpec((1, D), lambda i: (0, 0)),
            pl.BlockSpec((D, D), lambda i: (0, 0)),
            pl.BlockSpec((D, 2), lambda i: (0, 0)),
        ],
        out_specs=[
            pl.BlockSpec((B, 1), lambda i: (i, 0)),
            pl.BlockSpec((B, 1), lambda i: (i, 0)),
            pl.BlockSpec((B,), lambda i: (i,)),
        ],
        out_shape=[
            jax.ShapeDtypeStruct((NPAD, 1), jnp.float32),
            jax.ShapeDtypeStruct((NPAD, 1), jnp.float32),
            jax.ShapeDtypeStruct((NPAD,), jnp.float32),
        ],
    )(part, cntp, xp, w1lT, b1r, w1rT, w2)


def _sc_scalar_aggregate(p, spk, dpk):
    @functools.partial(
        pl.kernel,
        out_type=jax.ShapeDtypeStruct((NW, NPAD), jnp.float32),
        mesh=plsc.VectorSubcoreMesh(core_axis_name="c", subcore_axis_name="s"),
        compiler_params=pltpu.CompilerParams(needs_layout_passes=False),
        scratch_types=[
            pltpu.VMEM((NPAD,), jnp.float32),
            pltpu.VMEM((NPAD,), jnp.float32),
            pltpu.VMEM((EW // 2,), jnp.int32),
            pltpu.VMEM((EW // 2,), jnp.int32),
        ],
    )
    def k(p_hbm, spk_hbm, dpk_hbm, out_hbm, p_v, acc_v, spk_v, dpk_v):
        cid = lax.axis_index("c")
        sid = lax.axis_index("s")
        wid = cid * 16 + sid
        pltpu.sync_copy(p_hbm, p_v)
        pltpu.sync_copy(spk_hbm.at[wid], spk_v)
        pltpu.sync_copy(dpk_hbm.at[wid], dpk_v)
        zeros16 = jnp.zeros((16,), jnp.float32)

        def zacc(j, carry):
            acc_v[pl.ds(j * 16, 16)] = zeros16
            return carry

        lax.fori_loop(0, NPAD // 16, zacc, 0)

        def step(j, carry):
            sw = spk_v[pl.ds(j * 16, 16)]
            dw = dpk_v[pl.ds(j * 16, 16)]
            for part_ in range(2):
                if part_ == 0:
                    si = sw & 0xFFFF
                    dv = dw & 0xFFFF
                else:
                    si = lax.shift_right_logical(sw, 16)
                    dv = lax.shift_right_logical(dw, 16)
                vals = plsc.load_gather(p_v, [si])
                plsc.addupdate_scatter(acc_v, [dv], vals)
            return carry

        lax.fori_loop(0, EW // 32, step, 0)
        pltpu.sync_copy(acc_v, out_hbm.at[wid])

    return k(p, spk, dpk)


def _tc_final(pacc_r, cntc_r, q_r, noise_r, scal):
    def body(sc_ref, pacc_ref, cntc_ref, q_ref, noise_ref, out_ref):
        a = jnp.sum(pacc_ref[...], axis=0)
        h2 = a / cntc_ref[...] + sc_ref[0] + q_ref[...]
        out_ref[...] = (h2 * sc_ref[1] + sc_ref[2]
                        + noise_ref[...] * jnp.exp(h2 * sc_ref[3] + sc_ref[4]))

    R = NPAD // 128
    return pl.pallas_call(
        body,
        in_specs=[
            pl.BlockSpec(memory_space=pltpu.SMEM),
            pl.BlockSpec((NW, R, 128), lambda: (0, 0, 0)),
            pl.BlockSpec((R, 128), lambda: (0, 0)),
            pl.BlockSpec((R, 128), lambda: (0, 0)),
            pl.BlockSpec((R, 128), lambda: (0, 0)),
        ],
        out_specs=pl.BlockSpec((R, 128), lambda: (0, 0)),
        out_shape=jax.ShapeDtypeStruct((R, 128), jnp.float32),
    )(scal, pacc_r, cntc_r, q_r, noise_r)


def kernel(x, edge_index, W1l, b1, W1r, W2l, b2, W2r, Wal, ba, War, wm, bm, wv, bv):
    src = edge_index[0].astype(jnp.int32)
    dst = edge_index[1].astype(jnp.int32)
    pad_w = EW - E // NW
    pad_src = jnp.broadcast_to((jnp.arange(pad_w, dtype=jnp.int32) * 37) % N,
                               (NW, pad_w))
    pad_dst = jnp.broadcast_to(N + jnp.arange(pad_w, dtype=jnp.int32),
                               (NW, pad_w))
    srcp = jnp.concatenate([src.reshape(NW, E // NW), pad_src], axis=1)
    dstp = jnp.concatenate([dst.reshape(NW, E // NW), pad_dst], axis=1)

    def pack16(a):
        a16 = a.astype(jnp.int16).reshape(NW, EW // 2, 2)
        return lax.bitcast_convert_type(a16, jnp.int32)

    spk = pack16(srcp)
    dpk = pack16(dstp)
    part, cntp = _sc_aggregate(x, spk, dpk)

    xp = jnp.pad(x, ((0, NPAD - N), (0, 0)))
    w2 = jnp.stack([W2l[0], W2r[0]], axis=1)
    p2, q2, cntc = _tc_layer(part, cntp, xp, W1l.T, b1.reshape(1, D), W1r.T, w2)

    pacc = _sc_scalar_aggregate(p2.reshape(NPAD), spk, dpk)

    R = NPAD // 128
    noise = jax.random.normal(jax.random.key(42), (N, 1), jnp.float32)
    noise_r = jnp.pad(noise[:, 0], (0, NPAD - N)).reshape(R, 128)
    q_r = q2.reshape(R, 128)
    cntc_r = cntc.reshape(R, 128)
    pacc_r = pacc.reshape(NW, R, 128)
    scal = jnp.concatenate([b2, wm.ravel(), bm, wv.ravel(), bv])

    out_r = _tc_final(pacc_r, cntc_r, q_r, noise_r, scal)
    return out_r.reshape(NPAD)[:N][:, None]


# speedup vs baseline: 3.5122x; 1.2596x over previous
import functools

import jax
import jax.numpy as jnp
from jax import lax
from jax.experimental import pallas as pl
from jax.experimental.pallas import tpu as pltpu
from jax.experimental.pallas import tpu_sc as plsc

N = 10000
D = 128
NPAD = 10240
E = 320000
NW = 32
EW = 10240
EP = NW * EW
C = 64
CH = EW // C
NBUF = 2
RPT = NPAD // 16


def _sc_aggregate(x, spk, dpk):
    @functools.partial(
        pl.kernel,
        out_type=[
            jax.ShapeDtypeStruct((2, NPAD, D), jnp.float32),
            jax.ShapeDtypeStruct((NW, NPAD), jnp.float32),
        ],
        mesh=plsc.VectorSubcoreMesh(core_axis_name="c", subcore_axis_name="s"),
        compiler_params=pltpu.CompilerParams(needs_layout_passes=False),
        scratch_types=[
            pltpu.VMEM_SHARED((NPAD, D), jnp.float32),
            pltpu.VMEM((EW,), jnp.int32),
            pltpu.VMEM((EW,), jnp.int32),
            pltpu.VMEM((NBUF, C, D), jnp.float32),
            pltpu.VMEM((NPAD,), jnp.float32),
            pltpu.SemaphoreType.DMA((NBUF,)),
            pltpu.SemaphoreType.DMA((NBUF,)),
        ],
    )
    def k(x_hbm, src_hbm, dst_hbm, part_hbm, cnt_hbm,
          acc_sh, sidx_v, didx_v, rows, cnt_v, gsem, ssem):
        cid = lax.axis_index("c")
        sid = lax.axis_index("s")
        wid = cid * 16 + sid
        zeros16 = jnp.zeros((16,), jnp.float32)
        ones16 = jnp.ones((16,), jnp.float32)

        with jax.named_scope("zero_stage"):
            pltpu.sync_copy(src_hbm.at[wid], sidx_v)
            pltpu.sync_copy(dst_hbm.at[wid], didx_v)

            def zcnt(j, carry):
                cnt_v[pl.ds(j * 16, 16)] = zeros16
                return carry

            lax.fori_loop(0, NPAD // 16, zcnt, 0)

            def zrow(r, carry):
                for i in range(D // 16):
                    rows[0, r, pl.ds(i * 16, 16)] = zeros16
                return carry

            lax.fori_loop(0, C, zrow, 0)
            for kk in range(RPT // C):
                pltpu.sync_copy(rows.at[0],
                                acc_sh.at[pl.ds(sid * RPT + kk * C, C)])
            plsc.subcore_barrier()

        for b in range(NBUF):
            pltpu.async_copy(x_hbm.at[sidx_v.at[pl.ds(b * C, C)]],
                             rows.at[b], gsem.at[b])

        def outer(o, carry):
            for b in range(NBUF):
                g = o * NBUF + b
                pltpu.make_async_copy(
                    x_hbm.at[sidx_v.at[pl.ds(g * C, C)]],
                    rows.at[b], gsem.at[b]).wait()
                sc_d = pltpu.async_copy(
                    rows.at[b], acc_sh.at[didx_v.at[pl.ds(g * C, C)]],
                    ssem.at[b], add=True)
                for t in range(C // 16):
                    dv = didx_v[pl.ds(g * C + t * 16, 16)]
                    plsc.addupdate_scatter(cnt_v, [dv], ones16)
                sc_d.wait()

                @pl.when(g < CH - NBUF)
                def _():
                    pltpu.async_copy(
                        x_hbm.at[sidx_v.at[pl.ds((g + NBUF) * C, C)]],
                        rows.at[b], gsem.at[b])
            return carry

        with jax.named_scope("main_loop"):
            lax.fori_loop(0, CH // NBUF, outer, 0)
            plsc.subcore_barrier()

        with jax.named_scope("writeout"):
            for kk in range(RPT // C):
                r0 = sid * RPT + kk * C
                pltpu.sync_copy(acc_sh.at[pl.ds(r0, C)],
                                part_hbm.at[cid, pl.ds(r0, C)])
            pltpu.sync_copy(cnt_v, cnt_hbm.at[wid])

    return k(x, spk, dpk)


def _tc_layer(part, cntp, xp, w1lT, b1r, w1rT, w2):
    def body(part_ref, cnt_ref, x_ref, wl_ref, b1_ref, wr_ref, w2_ref,
             p_ref, q_ref, cntc_ref):
        s = part_ref[0] + part_ref[1]
        cntc = jnp.maximum(jnp.sum(cnt_ref[...], axis=0), 1.0)
        aggr = s / cntc[:, None]
        h = jnp.maximum(
            jnp.dot(aggr, wl_ref[...], preferred_element_type=jnp.float32)
            + b1_ref[...]
            + jnp.dot(x_ref[...], wr_ref[...], preferred_element_type=jnp.float32),
            0.0)
        pq = jnp.dot(h, w2_ref[...], preferred_element_type=jnp.float32)
        p_ref[...] = pq[:, 0:1]
        q_ref[...] = pq[:, 1:2]
        cntc_ref[...] = cntc

    B = 512
    grid = NPAD // B
    return pl.pallas_call(
        body,
        grid=(grid,),
        in_specs=[
            pl.BlockSpec((2, B, D), lambda i: (0, i, 0)),
            pl.BlockSpec((NW, B), lambda i: (0, i)),
            pl.BlockSpec((B, D), lambda i: (i, 0)),
            pl.BlockSpec((D, D), lambda i: (0, 0)),
            pl.BlockSpec((1, D), lambda i: (0, 0)),
            pl.BlockSpec((D, D), lambda i: (0, 0)),
            pl.BlockSpec((D, 2), lambda i: (0, 0)),
        ],
        out_specs=[
            pl.BlockSpec((B, 1), lambda i: (i, 0)),
            pl.BlockSpec((B, 1), lambda i: (i, 0)),
            pl.BlockSpec((B,), lambda i: (i,)),
        ],
        out_shape=[
            jax.ShapeDtypeStruct((NPAD, 1), jnp.float32),
            jax.ShapeDtypeStruct((NPAD, 1), jnp.float32),
            jax.ShapeDtypeStruct((NPAD,), jnp.float32),
        ],
    )(part, cntp, xp, w1lT, b1r, w1rT, w2)


def _sc_scalar_aggregate(p, srcw, dstw):
    @functools.partial(
        pl.kernel,
        out_type=jax.ShapeDtypeStruct((NW, NPAD), jnp.float32),
        mesh=plsc.VectorSubcoreMesh(core_axis_name="c", subcore_axis_name="s"),
        compiler_params=pltpu.CompilerParams(needs_layout_passes=False),
        scratch_types=[
            pltpu.VMEM((NPAD,), jnp.float32),
            pltpu.VMEM((NPAD,), jnp.float32),
            pltpu.VMEM((EW,), jnp.int32),
            pltpu.VMEM((EW,), jnp.int32),
        ],
    )
    def k(p_hbm, src_hbm, dst_hbm, out_hbm, p_v, acc_v, sidx_v, didx_v):
        cid = lax.axis_index("c")
        sid = lax.axis_index("s")
        wid = cid * 16 + sid
        pltpu.sync_copy(p_hbm, p_v)
        pltpu.sync_copy(src_hbm.at[wid], sidx_v)
        pltpu.sync_copy(dst_hbm.at[wid], didx_v)
        zeros16 = jnp.zeros((16,), jnp.float32)

        def zacc(j, carry):
            acc_v[pl.ds(j * 16, 16)] = zeros16
            return carry

        lax.fori_loop(0, NPAD // 16, zacc, 0)

        def step(j, carry):
            si = sidx_v[pl.ds(j * 16, 16)]
            dv = didx_v[pl.ds(j * 16, 16)]
            vals = plsc.load_gather(p_v, [si])
            plsc.addupdate_scatter(acc_v, [dv], vals)
            return carry

        lax.fori_loop(0, EW // 16, step, 0)
        pltpu.sync_copy(acc_v, out_hbm.at[wid])

    return k(p, srcw, dstw)


def _tc_final(pacc_r, cntc_r, q_r, noise_r, scal):
    def body(sc_ref, pacc_ref, cntc_ref, q_ref, noise_ref, out_ref):
        a = jnp.sum(pacc_ref[...], axis=0)
        h2 = a / cntc_ref[...] + sc_ref[0] + q_ref[...]
        out_ref[...] = (h2 * sc_ref[1] + sc_ref[2]
                        + noise_ref[...] * jnp.exp(h2 * sc_ref[3] + sc_ref[4]))

    R = NPAD // 128
    return pl.pallas_call(
        body,
        in_specs=[
            pl.BlockSpec(memory_space=pltpu.SMEM),
            pl.BlockSpec((NW, R, 128), lambda: (0, 0, 0)),
            pl.BlockSpec((R, 128), lambda: (0, 0)),
            pl.BlockSpec((R, 128), lambda: (0, 0)),
            pl.BlockSpec((R, 128), lambda: (0, 0)),
        ],
        out_specs=pl.BlockSpec((R, 128), lambda: (0, 0)),
        out_shape=jax.ShapeDtypeStruct((R, 128), jnp.float32),
    )(scal, pacc_r, cntc_r, q_r, noise_r)


def kernel(x, edge_index, W1l, b1, W1r, W2l, b2, W2r, Wal, ba, War, wm, bm, wv, bv):
    src = edge_index[0].astype(jnp.int32)
    dst = edge_index[1].astype(jnp.int32)
    pad_w = EW - E // NW
    pad_src = jnp.broadcast_to((jnp.arange(pad_w, dtype=jnp.int32) * 37) % N,
                               (NW, pad_w))
    pad_dst = jnp.broadcast_to(N + jnp.arange(pad_w, dtype=jnp.int32),
                               (NW, pad_w))
    srcp = jnp.concatenate([src.reshape(NW, E // NW), pad_src], axis=1)
    dstp = jnp.concatenate([dst.reshape(NW, E // NW), pad_dst], axis=1)

    part, cntp = _sc_aggregate(x, srcp, dstp)

    xp = jnp.pad(x, ((0, NPAD - N), (0, 0)))
    w2 = jnp.stack([W2l[0], W2r[0]], axis=1)
    p2, q2, cntc = _tc_layer(part, cntp, xp, W1l.T, b1.reshape(1, D), W1r.T, w2)

    pacc = _sc_scalar_aggregate(p2.reshape(NPAD), srcp, dstp)

    R = NPAD // 128
    noise = jax.random.normal(jax.random.key(42), (N, 1), jnp.float32)
    noise_r = jnp.pad(noise[:, 0], (0, NPAD - N)).reshape(R, 128)
    q_r = q2.reshape(R, 128)
    cntc_r = cntc.reshape(R, 128)
    pacc_r = pacc.reshape(NW, R, 128)
    scal = jnp.concatenate([b2, wm.ravel(), bm, wv.ravel(), bv])

    out_r = _tc_final(pacc_r, cntc_r, q_r, noise_r, scal)
    return out_r.reshape(NPAD)[:N][:, None]


# speedup vs baseline: 3.7821x; 1.0769x over previous
import functools

import jax
import jax.numpy as jnp
from jax import lax
from jax.experimental import pallas as pl
from jax.experimental.pallas import tpu as pltpu
from jax.experimental.pallas import tpu_sc as plsc

N = 10000
D = 128
NPAD = 10240
E = 320000
NW = 32
EW = 10240
EWR = E // NW
PAD_W = EW - EWR
EP = NW * EW
C = 64
CH = EW // C
NBUF = 2
RPT = NPAD // 16


def _sc_aggregate(x, spk, dpk):
    @functools.partial(
        pl.kernel,
        out_type=[
            jax.ShapeDtypeStruct((2, NPAD, D), jnp.float32),
            jax.ShapeDtypeStruct((NW, NPAD), jnp.float32),
        ],
        mesh=plsc.VectorSubcoreMesh(core_axis_name="c", subcore_axis_name="s"),
        compiler_params=pltpu.CompilerParams(needs_layout_passes=False),
        scratch_types=[
            pltpu.VMEM_SHARED((NPAD, D), jnp.float32),
            pltpu.VMEM((EW,), jnp.int32),
            pltpu.VMEM((EW,), jnp.int32),
            pltpu.VMEM((NBUF, C, D), jnp.float32),
            pltpu.VMEM((NPAD,), jnp.float32),
            pltpu.SemaphoreType.DMA((NBUF,)),
            pltpu.SemaphoreType.DMA((NBUF,)),
        ],
    )
    def k(x_hbm, src_hbm, dst_hbm, part_hbm, cnt_hbm,
          acc_sh, sidx_v, didx_v, rows, cnt_v, gsem, ssem):
        cid = lax.axis_index("c")
        sid = lax.axis_index("s")
        wid = cid * 16 + sid
        zeros16 = jnp.zeros((16,), jnp.float32)
        ones16 = jnp.ones((16,), jnp.float32)

        with jax.named_scope("zero_stage"):
            pltpu.sync_copy(src_hbm.at[pl.ds(wid * EWR, EWR)],
                            sidx_v.at[pl.ds(0, EWR)])
            pltpu.sync_copy(dst_hbm.at[pl.ds(wid * EWR, EWR)],
                            didx_v.at[pl.ds(0, EWR)])

            def padfill(j, carry):
                base = jnp.full((16,), j * 16, jnp.int32) + lax.iota(jnp.int32, 16)
                sidx_v[pl.ds(EWR + j * 16, 16)] = base
                didx_v[pl.ds(EWR + j * 16, 16)] = base + N
                return carry

            lax.fori_loop(0, PAD_W // 16, padfill, 0)

            def zcnt(j, carry):
                cnt_v[pl.ds(j * 16, 16)] = zeros16
                return carry

            lax.fori_loop(0, NPAD // 16, zcnt, 0)

            def zrow(r, carry):
                for i in range(D // 16):
                    rows[0, r, pl.ds(i * 16, 16)] = zeros16
                return carry

            lax.fori_loop(0, C, zrow, 0)
            for kk in range(RPT // C):
                pltpu.sync_copy(rows.at[0],
                                acc_sh.at[pl.ds(sid * RPT + kk * C, C)])
            plsc.subcore_barrier()

        for b in range(NBUF):
            pltpu.async_copy(x_hbm.at[sidx_v.at[pl.ds(b * C, C)]],
                             rows.at[b], gsem.at[b])

        def outer(o, carry):
            for b in range(NBUF):
                g = o * NBUF + b
                pltpu.make_async_copy(
                    x_hbm.at[sidx_v.at[pl.ds(g * C, C)]],
                    rows.at[b], gsem.at[b]).wait()
                sc_d = pltpu.async_copy(
                    rows.at[b], acc_sh.at[didx_v.at[pl.ds(g * C, C)]],
                    ssem.at[b], add=True)
                for t in range(C // 16):
                    dv = didx_v[pl.ds(g * C + t * 16, 16)]
                    plsc.addupdate_scatter(cnt_v, [dv], ones16)
                sc_d.wait()

                @pl.when(g < CH - NBUF)
                def _():
                    pltpu.async_copy(
                        x_hbm.at[sidx_v.at[pl.ds((g + NBUF) * C, C)]],
                        rows.at[b], gsem.at[b])
            return carry

        with jax.named_scope("main_loop"):
            lax.fori_loop(0, CH // NBUF, outer, 0)
            plsc.subcore_barrier()

        with jax.named_scope("writeout"):
            for kk in range(RPT // C):
                r0 = sid * RPT + kk * C
                pltpu.sync_copy(acc_sh.at[pl.ds(r0, C)],
                                part_hbm.at[cid, pl.ds(r0, C)])
            pltpu.sync_copy(cnt_v, cnt_hbm.at[wid])

    return k(x, spk, dpk)


def _tc_layer(part, cntp, xp, w1lT, b1r, w1rT, w2):
    def body(part_ref, cnt_ref, x_ref, wl_ref, b1_ref, wr_ref, w2_ref,
             p_ref, q_ref, cntc_ref):
        s = part_ref[0] + part_ref[1]
        cntc = jnp.maximum(jnp.sum(cnt_ref[...], axis=0), 1.0)
        aggr = s / cntc[:, None]
        h = jnp.maximum(
            jnp.dot(aggr, wl_ref[...], preferred_element_type=jnp.float32)
            + b1_ref[...]
            + jnp.dot(x_ref[...], wr_ref[...], preferred_element_type=jnp.float32),
            0.0)
        pq = jnp.dot(h, w2_ref[...], preferred_element_type=jnp.float32)
        p_ref[...] = pq[:, 0:1]
        q_ref[...] = pq[:, 1:2]
        cntc_ref[...] = cntc

    B = 512
    grid = NPAD // B
    return pl.pallas_call(
        body,
        grid=(grid,),
        in_specs=[
            pl.BlockSpec((2, B, D), lambda i: (0, i, 0)),
            pl.BlockSpec((NW, B), lambda i: (0, i)),
            pl.BlockSpec((B, D), lambda i: (i, 0)),
            pl.BlockSpec((D, D), lambda i: (0, 0)),
            pl.BlockSpec((1, D), lambda i: (0, 0)),
            pl.BlockSpec((D, D), lambda i: (0, 0)),
            pl.BlockSpec((D, 2), lambda i: (0, 0)),
        ],
        out_specs=[
            pl.BlockSpec((B, 1), lambda i: (i, 0)),
            pl.BlockSpec((B, 1), lambda i: (i, 0)),
            pl.BlockSpec((B,), lambda i: (i,)),
        ],
        out_shape=[
            jax.ShapeDtypeStruct((NPAD, 1), jnp.float32),
            jax.ShapeDtypeStruct((NPAD, 1), jnp.float32),
            jax.ShapeDtypeStruct((NPAD,), jnp.float32),
        ],
    )(part, cntp, xp, w1lT, b1r, w1rT, w2)


def _sc_scalar_aggregate(p, srcw, dstw):
    @functools.partial(
        pl.kernel,
        out_type=jax.ShapeDtypeStruct((NW, NPAD), jnp.float32),
        mesh=plsc.VectorSubcoreMesh(core_axis_name="c", subcore_axis_name="s"),
        compiler_params=pltpu.CompilerParams(needs_layout_passes=False),
        scratch_types=[
            pltpu.VMEM((NPAD,), jnp.float32),
            pltpu.VMEM((NPAD,), jnp.float32),
            pltpu.VMEM((EW,), jnp.int32),
            pltpu.VMEM((EW,), jnp.int32),
        ],
    )
    def k(p_hbm, src_hbm, dst_hbm, out_hbm, p_v, acc_v, sidx_v, didx_v):
        cid = lax.axis_index("c")
        sid = lax.axis_index("s")
        wid = cid * 16 + sid
        pltpu.sync_copy(p_hbm, p_v)
        pltpu.sync_copy(src_hbm.at[pl.ds(wid * EWR, EWR)],
                        sidx_v.at[pl.ds(0, EWR)])
        pltpu.sync_copy(dst_hbm.at[pl.ds(wid * EWR, EWR)],
                        didx_v.at[pl.ds(0, EWR)])
        zeros16 = jnp.zeros((16,), jnp.float32)

        def padfill(j, carry):
            base = jnp.full((16,), j * 16, jnp.int32) + lax.iota(jnp.int32, 16)
            sidx_v[pl.ds(EWR + j * 16, 16)] = base
            didx_v[pl.ds(EWR + j * 16, 16)] = base + N
            return carry

        lax.fori_loop(0, PAD_W // 16, padfill, 0)

        def zacc(j, carry):
            acc_v[pl.ds(j * 16, 16)] = zeros16
            return carry

        lax.fori_loop(0, NPAD // 16, zacc, 0)

        def step(j, carry):
            si = sidx_v[pl.ds(j * 16, 16)]
            dv = didx_v[pl.ds(j * 16, 16)]
            vals = plsc.load_gather(p_v, [si])
            plsc.addupdate_scatter(acc_v, [dv], vals)
            return carry

        lax.fori_loop(0, EW // 16, step, 0)
        pltpu.sync_copy(acc_v, out_hbm.at[wid])

    return k(p, srcw, dstw)


def _tc_final(pacc_r, cntc_r, q_r, noise_r, scal):
    def body(sc_ref, pacc_ref, cntc_ref, q_ref, noise_ref, out_ref):
        a = jnp.sum(pacc_ref[...], axis=0)
        h2 = a / cntc_ref[...] + sc_ref[0] + q_ref[...]
        out_ref[...] = (h2 * sc_ref[1] + sc_ref[2]
                        + noise_ref[...] * jnp.exp(h2 * sc_ref[3] + sc_ref[4]))

    R = NPAD // 128
    return pl.pallas_call(
        body,
        in_specs=[
            pl.BlockSpec(memory_space=pltpu.SMEM),
            pl.BlockSpec((NW, R, 128), lambda: (0, 0, 0)),
            pl.BlockSpec((R, 128), lambda: (0, 0)),
            pl.BlockSpec((R, 128), lambda: (0, 0)),
            pl.BlockSpec((R, 128), lambda: (0, 0)),
        ],
        out_specs=pl.BlockSpec((R, 128), lambda: (0, 0)),
        out_shape=jax.ShapeDtypeStruct((R, 128), jnp.float32),
    )(scal, pacc_r, cntc_r, q_r, noise_r)


def kernel(x, edge_index, W1l, b1, W1r, W2l, b2, W2r, Wal, ba, War, wm, bm, wv, bv):
    src = edge_index[0].astype(jnp.int32)
    dst = edge_index[1].astype(jnp.int32)

    part, cntp = _sc_aggregate(x, src, dst)

    xp = jnp.pad(x, ((0, NPAD - N), (0, 0)))
    w2 = jnp.stack([W2l[0], W2r[0]], axis=1)
    p2, q2, cntc = _tc_layer(part, cntp, xp, W1l.T, b1.reshape(1, D), W1r.T, w2)

    pacc = _sc_scalar_aggregate(p2.reshape(NPAD), src, dst)

    R = NPAD // 128
    noise = jax.random.normal(jax.random.key(42), (N, 1), jnp.float32)
    noise_r = jnp.pad(noise[:, 0], (0, NPAD - N)).reshape(R, 128)
    q_r = q2.reshape(R, 128)
    cntc_r = cntc.reshape(R, 128)
    pacc_r = pacc.reshape(NW, R, 128)
    scal = jnp.concatenate([b2, wm.ravel(), bm, wv.ravel(), bv])

    out_r = _tc_final(pacc_r, cntc_r, q_r, noise_r, scal)
    return out_r.reshape(NPAD)[:N][:, None]


# speedup vs baseline: 3.7878x; 1.0015x over previous
import functools

import jax
import jax.numpy as jnp
from jax import lax
from jax.experimental import pallas as pl
from jax.experimental.pallas import tpu as pltpu
from jax.experimental.pallas import tpu_sc as plsc

N = 10000
D = 128
NPAD = 10240
E = 320000
NW = 32
EW = 10240
EWR = E // NW
PAD_W = EW - EWR
EP = NW * EW
C = 64
CH = EW // C
NBUF = 2
RPT = NPAD // 16


def _sc_aggregate(x, spk, dpk):
    @functools.partial(
        pl.kernel,
        out_type=[
            jax.ShapeDtypeStruct((2, NPAD, D), jnp.float32),
            jax.ShapeDtypeStruct((NW, NPAD), jnp.float32),
        ],
        mesh=plsc.VectorSubcoreMesh(core_axis_name="c", subcore_axis_name="s"),
        compiler_params=pltpu.CompilerParams(needs_layout_passes=False),
        scratch_types=[
            pltpu.VMEM_SHARED((NPAD, D), jnp.float32),
            pltpu.VMEM((EW,), jnp.int32),
            pltpu.VMEM((EW,), jnp.int32),
            pltpu.VMEM((NBUF, C, D), jnp.float32),
            pltpu.VMEM((NPAD,), jnp.float32),
            pltpu.SemaphoreType.DMA((NBUF,)),
            pltpu.SemaphoreType.DMA((NBUF,)),
        ],
    )
    def k(x_hbm, src_hbm, dst_hbm, part_hbm, cnt_hbm,
          acc_sh, sidx_v, didx_v, rows, cnt_v, gsem, ssem):
        cid = lax.axis_index("c")
        sid = lax.axis_index("s")
        wid = cid * 16 + sid
        zeros16 = jnp.zeros((16,), jnp.float32)
        ones16 = jnp.ones((16,), jnp.float32)

        with jax.named_scope("zero_stage"):
            pltpu.sync_copy(src_hbm.at[pl.ds(wid * EWR, EWR)],
                            sidx_v.at[pl.ds(0, EWR)])
            pltpu.sync_copy(dst_hbm.at[pl.ds(wid * EWR, EWR)],
                            didx_v.at[pl.ds(0, EWR)])

            def padfill(j, carry):
                base = jnp.full((16,), j * 16, jnp.int32) + lax.iota(jnp.int32, 16)
                sidx_v[pl.ds(EWR + j * 16, 16)] = base
                didx_v[pl.ds(EWR + j * 16, 16)] = base + N
                return carry

            lax.fori_loop(0, PAD_W // 16, padfill, 0)

            def zcnt(j, carry):
                cnt_v[pl.ds(j * 16, 16)] = zeros16
                return carry

            lax.fori_loop(0, NPAD // 16, zcnt, 0)

            def zrow(r, carry):
                for i in range(D // 16):
                    rows[0, r, pl.ds(i * 16, 16)] = zeros16
                return carry

            lax.fori_loop(0, C, zrow, 0)
            for kk in range(RPT // C):
                pltpu.sync_copy(rows.at[0],
                                acc_sh.at[pl.ds(sid * RPT + kk * C, C)])
            plsc.subcore_barrier()

        for b in range(NBUF):
            pltpu.async_copy(x_hbm.at[sidx_v.at[pl.ds(b * C, C)]],
                             rows.at[b], gsem.at[b])

        def outer(o, carry):
            for b in range(NBUF):
                g = o * NBUF + b
                pltpu.make_async_copy(
                    x_hbm.at[sidx_v.at[pl.ds(g * C, C)]],
                    rows.at[b], gsem.at[b]).wait()
                sc_d = pltpu.async_copy(
                    rows.at[b], acc_sh.at[didx_v.at[pl.ds(g * C, C)]],
                    ssem.at[b], add=True)
                for t in range(C // 16):
                    dv = didx_v[pl.ds(g * C + t * 16, 16)]
                    plsc.addupdate_scatter(cnt_v, [dv], ones16)
                sc_d.wait()

                @pl.when(g < CH - NBUF)
                def _():
                    pltpu.async_copy(
                        x_hbm.at[sidx_v.at[pl.ds((g + NBUF) * C, C)]],
                        rows.at[b], gsem.at[b])
            return carry

        with jax.named_scope("main_loop"):
            lax.fori_loop(0, CH // NBUF, outer, 0)
            plsc.subcore_barrier()

        with jax.named_scope("writeout"):
            for kk in range(RPT // C):
                r0 = sid * RPT + kk * C
                pltpu.sync_copy(acc_sh.at[pl.ds(r0, C)],
                                part_hbm.at[cid, pl.ds(r0, C)])
            pltpu.sync_copy(cnt_v, cnt_hbm.at[wid])

    return k(x, spk, dpk)


def _tc_xr(xp, w1rT, b1r):
    def body(x_ref, wr_ref, b1_ref, xr_ref):
        xr_ref[...] = (
            jnp.dot(x_ref[...], wr_ref[...], preferred_element_type=jnp.float32)
            + b1_ref[...])

    B = 512
    return pl.pallas_call(
        body,
        grid=(NPAD // B,),
        in_specs=[
            pl.BlockSpec((B, D), lambda i: (i, 0)),
            pl.BlockSpec((D, D), lambda i: (0, 0)),
            pl.BlockSpec((1, D), lambda i: (0, 0)),
        ],
        out_specs=pl.BlockSpec((B, D), lambda i: (i, 0)),
        out_shape=jax.ShapeDtypeStruct((NPAD, D), jnp.float32),
    )(xp, w1rT, b1r)


def _tc_layer(part, cntp, xr, w1lT, w2):
    def body(part_ref, cnt_ref, xr_ref, wl_ref, w2_ref,
             p_ref, q_ref, cntc_ref):
        s = part_ref[0] + part_ref[1]
        cntc = jnp.maximum(jnp.sum(cnt_ref[...], axis=0), 1.0)
        aggr = s / cntc[:, None]
        h = jnp.maximum(
            jnp.dot(aggr, wl_ref[...], preferred_element_type=jnp.float32)
            + xr_ref[...],
            0.0)
        pq = jnp.dot(h, w2_ref[...], preferred_element_type=jnp.float32)
        p_ref[...] = pq[:, 0:1]
        q_ref[...] = pq[:, 1:2]
        cntc_ref[...] = cntc

    B = 512
    grid = NPAD // B
    return pl.pallas_call(
        body,
        grid=(grid,),
        in_specs=[
            pl.BlockSpec((2, B, D), lambda i: (0, i, 0)),
            pl.BlockSpec((NW, B), lambda i: (0, i)),
            pl.BlockSpec((B, D), lambda i: (i, 0)),
            pl.BlockSpec((D, D), lambda i: (0, 0)),
            pl.BlockSpec((D, 2), lambda i: (0, 0)),
        ],
        out_specs=[
            pl.BlockSpec((B, 1), lambda i: (i, 0)),
            pl.BlockSpec((B, 1), lambda i: (i, 0)),
            pl.BlockSpec((B,), lambda i: (i,)),
        ],
        out_shape=[
            jax.ShapeDtypeStruct((NPAD, 1), jnp.float32),
            jax.ShapeDtypeStruct((NPAD, 1), jnp.float32),
            jax.ShapeDtypeStruct((NPAD,), jnp.float32),
        ],
    )(part, cntp, xr, w1lT, w2)


def _sc_scalar_aggregate(p, srcw, dstw):
    @functools.partial(
        pl.kernel,
        out_type=jax.ShapeDtypeStruct((NW, NPAD), jnp.float32),
        mesh=plsc.VectorSubcoreMesh(core_axis_name="c", subcore_axis_name="s"),
        compiler_params=pltpu.CompilerParams(needs_layout_passes=False),
        scratch_types=[
            pltpu.VMEM((NPAD,), jnp.float32),
            pltpu.VMEM((NPAD,), jnp.float32),
            pltpu.VMEM((EW,), jnp.int32),
            pltpu.VMEM((EW,), jnp.int32),
        ],
    )
    def k(p_hbm, src_hbm, dst_hbm, out_hbm, p_v, acc_v, sidx_v, didx_v):
        cid = lax.axis_index("c")
        sid = lax.axis_index("s")
        wid = cid * 16 + sid
        pltpu.sync_copy(p_hbm, p_v)
        pltpu.sync_copy(src_hbm.at[pl.ds(wid * EWR, EWR)],
                        sidx_v.at[pl.ds(0, EWR)])
        pltpu.sync_copy(dst_hbm.at[pl.ds(wid * EWR, EWR)],
                        didx_v.at[pl.ds(0, EWR)])
        zeros16 = jnp.zeros((16,), jnp.float32)

        def padfill(j, carry):
            base = jnp.full((16,), j * 16, jnp.int32) + lax.iota(jnp.int32, 16)
            sidx_v[pl.ds(EWR + j * 16, 16)] = base
            didx_v[pl.ds(EWR + j * 16, 16)] = base + N
            return carry

        lax.fori_loop(0, PAD_W // 16, padfill, 0)

        def zacc(j, carry):
            acc_v[pl.ds(j * 16, 16)] = zeros16
            return carry

        lax.fori_loop(0, NPAD // 16, zacc, 0)

        def step(j, carry):
            si = sidx_v[pl.ds(j * 16, 16)]
            dv = didx_v[pl.ds(j * 16, 16)]
            vals = plsc.load_gather(p_v, [si])
            plsc.addupdate_scatter(acc_v, [dv], vals)
            return carry

        lax.fori_loop(0, EW // 16, step, 0)
        pltpu.sync_copy(acc_v, out_hbm.at[wid])

    return k(p, srcw, dstw)


def _tc_final(pacc_r, cntc_r, q_r, noise_r, scal):
    def body(sc_ref, pacc_ref, cntc_ref, q_ref, noise_ref, out_ref):
        a = jnp.sum(pacc_ref[...], axis=0)
        h2 = a / cntc_ref[...] + sc_ref[0] + q_ref[...]
        out_ref[...] = (h2 * sc_ref[1] + sc_ref[2]
                        + noise_ref[...] * jnp.exp(h2 * sc_ref[3] + sc_ref[4]))

    R = NPAD // 128
    return pl.pallas_call(
        body,
        in_specs=[
            pl.BlockSpec(memory_space=pltpu.SMEM),
            pl.BlockSpec((NW, R, 128), lambda: (0, 0, 0)),
            pl.BlockSpec((R, 128), lambda: (0, 0)),
            pl.BlockSpec((R, 128), lambda: (0, 0)),
            pl.BlockSpec((R, 128), lambda: (0, 0)),
        ],
        out_specs=pl.BlockSpec((R, 128), lambda: (0, 0)),
        out_shape=jax.ShapeDtypeStruct((R, 128), jnp.float32),
    )(scal, pacc_r, cntc_r, q_r, noise_r)


def kernel(x, edge_index, W1l, b1, W1r, W2l, b2, W2r, Wal, ba, War, wm, bm, wv, bv):
    src = edge_index[0].astype(jnp.int32)
    dst = edge_index[1].astype(jnp.int32)

    part, cntp = _sc_aggregate(x, src, dst)

    xp = jnp.pad(x, ((0, NPAD - N), (0, 0)))
    w2 = jnp.stack([W2l[0], W2r[0]], axis=1)
    xr = _tc_xr(xp, W1r.T, b1.reshape(1, D))
    p2, q2, cntc = _tc_layer(part, cntp, xr, W1l.T, w2)

    pacc = _sc_scalar_aggregate(p2.reshape(NPAD), src, dst)

    R = NPAD // 128
    noise = jax.random.normal(jax.random.key(42), (N, 1), jnp.float32)
    noise_r = jnp.pad(noise[:, 0], (0, NPAD - N)).reshape(R, 128)
    q_r = q2.reshape(R, 128)
    cntc_r = cntc.reshape(R, 128)
    pacc_r = pacc.reshape(NW, R, 128)
    scal = jnp.concatenate([b2, wm.ravel(), bm, wv.ravel(), bv])

    out_r = _tc_final(pacc_r, cntc_r, q_r, noise_r, scal)
    return out_r.reshape(NPAD)[:N][:, None]


# speedup vs baseline: 3.7891x; 1.0004x over previous
import functools

import jax
import jax.numpy as jnp
from jax import lax
from jax.experimental import pallas as pl
from jax.experimental.pallas import tpu as pltpu
from jax.experimental.pallas import tpu_sc as plsc

N = 10000
D = 128
NPAD = 10240
E = 320000
NW = 32
EW = 10240
EWR = E // NW
PAD_W = EW - EWR
EP = NW * EW
C = 64
CH = EW // C
NBUF = 2
RPT = NPAD // 16


def _sc_aggregate(x, src, dst):
    @functools.partial(
        pl.kernel,
        out_type=[
            jax.ShapeDtypeStruct((2, NPAD, D), jnp.float32),
            jax.ShapeDtypeStruct((NW, NPAD), jnp.float32),
        ],
        mesh=plsc.VectorSubcoreMesh(core_axis_name="c", subcore_axis_name="s"),
        compiler_params=pltpu.CompilerParams(needs_layout_passes=False),
        scratch_types=[
            pltpu.VMEM_SHARED((NPAD, D), jnp.float32),
            pltpu.VMEM((EW,), jnp.int32),
            pltpu.VMEM((EW,), jnp.int32),
            pltpu.VMEM((NBUF, C, D), jnp.float32),
            pltpu.VMEM((NPAD,), jnp.float32),
            pltpu.SemaphoreType.DMA((NBUF,)),
            pltpu.SemaphoreType.DMA((NBUF,)),
        ],
    )
    def k(x_hbm, src_hbm, dst_hbm, part_hbm, cnt_hbm,
          acc_sh, sidx_v, didx_v, rows, cnt_v, gsem, ssem):
        cid = lax.axis_index("c")
        sid = lax.axis_index("s")
        wid = cid * 16 + sid
        zeros16 = jnp.zeros((16,), jnp.float32)
        ones16 = jnp.ones((16,), jnp.float32)

        with jax.named_scope("zero_stage"):
            pltpu.sync_copy(src_hbm.at[pl.ds(wid * EWR, EWR)],
                            sidx_v.at[pl.ds(0, EWR)])
            pltpu.sync_copy(dst_hbm.at[pl.ds(wid * EWR, EWR)],
                            didx_v.at[pl.ds(0, EWR)])

            def padfill(j, carry):
                base = jnp.full((16,), j * 16, jnp.int32) + lax.iota(jnp.int32, 16)
                sidx_v[pl.ds(EWR + j * 16, 16)] = base
                didx_v[pl.ds(EWR + j * 16, 16)] = base + N
                return carry

            lax.fori_loop(0, PAD_W // 16, padfill, 0)

            def zcnt(j, carry):
                cnt_v[pl.ds(j * 16, 16)] = zeros16
                return carry

            lax.fori_loop(0, NPAD // 16, zcnt, 0)

            def zrow(r, carry):
                for i in range(D // 16):
                    rows[0, r, pl.ds(i * 16, 16)] = zeros16
                return carry

            lax.fori_loop(0, C, zrow, 0)
            for kk in range(RPT // C):
                pltpu.sync_copy(rows.at[0],
                                acc_sh.at[pl.ds(sid * RPT + kk * C, C)])
            plsc.subcore_barrier()

        for b in range(NBUF):
            pltpu.async_copy(x_hbm.at[sidx_v.at[pl.ds(b * C, C)]],
                             rows.at[b], gsem.at[b])

        def outer(o, carry):
            for b in range(NBUF):
                g = o * NBUF + b
                pltpu.make_async_copy(
                    x_hbm.at[sidx_v.at[pl.ds(g * C, C)]],
                    rows.at[b], gsem.at[b]).wait()
                sc_d = pltpu.async_copy(
                    rows.at[b], acc_sh.at[didx_v.at[pl.ds(g * C, C)]],
                    ssem.at[b], add=True)
                for t in range(C // 16):
                    dv = didx_v[pl.ds(g * C + t * 16, 16)]
                    plsc.addupdate_scatter(cnt_v, [dv], ones16)
                sc_d.wait()

                @pl.when(g < CH - NBUF)
                def _():
                    pltpu.async_copy(
                        x_hbm.at[sidx_v.at[pl.ds((g + NBUF) * C, C)]],
                        rows.at[b], gsem.at[b])
            return carry

        with jax.named_scope("main_loop"):
            lax.fori_loop(0, CH // NBUF, outer, 0)
            plsc.subcore_barrier()

        with jax.named_scope("writeout"):
            for kk in range(RPT // C):
                r0 = sid * RPT + kk * C
                pltpu.sync_copy(acc_sh.at[pl.ds(r0, C)],
                                part_hbm.at[cid, pl.ds(r0, C)])
            pltpu.sync_copy(cnt_v, cnt_hbm.at[wid])

    return k(x, src, dst)


def _tc_xr(xp, w1rT, b1r):
    def body(x_ref, wr_ref, b1_ref, xr_ref):
        xr_ref[...] = (
            jnp.dot(x_ref[...], wr_ref[...], preferred_element_type=jnp.float32)
            + b1_ref[...])

    B = 512
    return pl.pallas_call(
        body,
        grid=(NPAD // B,),
        in_specs=[
            pl.BlockSpec((B, D), lambda i: (i, 0)),
            pl.BlockSpec((D, D), lambda i: (0, 0)),
            pl.BlockSpec((1, D), lambda i: (0, 0)),
        ],
        out_specs=pl.BlockSpec((B, D), lambda i: (i, 0)),
        out_shape=jax.ShapeDtypeStruct((NPAD, D), jnp.float32),
    )(xp, w1rT, b1r)


def _tc_layer(part, cntp, xr, w1lT, w2):
    def body(part_ref, cnt_ref, xr_ref, wl_ref, w2_ref,
             p_ref, q_ref, cntc_ref):
        s = part_ref[0] + part_ref[1]
        cntc = jnp.maximum(jnp.sum(cnt_ref[...], axis=0), 1.0)
        aggr = s / cntc[:, None]
        h = jnp.maximum(
            jnp.dot(aggr, wl_ref[...], preferred_element_type=jnp.float32)
            + xr_ref[...],
            0.0)
        pq = jnp.dot(h, w2_ref[...], preferred_element_type=jnp.float32)
        p_ref[...] = pq[:, 0:1]
        q_ref[...] = pq[:, 1:2]
        cntc_ref[...] = cntc

    B = 512
    grid = NPAD // B
    return pl.pallas_call(
        body,
        grid=(grid,),
        in_specs=[
            pl.BlockSpec((2, B, D), lambda i: (0, i, 0)),
            pl.BlockSpec((NW, B), lambda i: (0, i)),
            pl.BlockSpec((B, D), lambda i: (i, 0)),
            pl.BlockSpec((D, D), lambda i: (0, 0)),
            pl.BlockSpec((D, 2), lambda i: (0, 0)),
        ],
        out_specs=[
            pl.BlockSpec((B, 1), lambda i: (i, 0)),
            pl.BlockSpec((B, 1), lambda i: (i, 0)),
            pl.BlockSpec((B,), lambda i: (i,)),
        ],
        out_shape=[
            jax.ShapeDtypeStruct((NPAD, 1), jnp.float32),
            jax.ShapeDtypeStruct((NPAD, 1), jnp.float32),
            jax.ShapeDtypeStruct((NPAD,), jnp.float32),
        ],
    )(part, cntp, xr, w1lT, w2)


def _sc_scalar_aggregate(p, srcw, dstw):
    @functools.partial(
        pl.kernel,
        out_type=jax.ShapeDtypeStruct((NW, NPAD), jnp.float32),
        mesh=plsc.VectorSubcoreMesh(core_axis_name="c", subcore_axis_name="s"),
        compiler_params=pltpu.CompilerParams(needs_layout_passes=False),
        scratch_types=[
            pltpu.VMEM((NPAD,), jnp.float32),
            pltpu.VMEM((NPAD,), jnp.float32),
            pltpu.VMEM((EW,), jnp.int32),
            pltpu.VMEM((EW,), jnp.int32),
        ],
    )
    def k(p_hbm, src_hbm, dst_hbm, out_hbm, p_v, acc_v, sidx_v, didx_v):
        cid = lax.axis_index("c")
        sid = lax.axis_index("s")
        wid = cid * 16 + sid
        pltpu.sync_copy(p_hbm, p_v)
        pltpu.sync_copy(src_hbm.at[pl.ds(wid * EWR, EWR)],
                        sidx_v.at[pl.ds(0, EWR)])
        pltpu.sync_copy(dst_hbm.at[pl.ds(wid * EWR, EWR)],
                        didx_v.at[pl.ds(0, EWR)])
        zeros16 = jnp.zeros((16,), jnp.float32)

        def padfill(j, carry):
            base = jnp.full((16,), j * 16, jnp.int32) + lax.iota(jnp.int32, 16)
            sidx_v[pl.ds(EWR + j * 16, 16)] = base
            didx_v[pl.ds(EWR + j * 16, 16)] = base + N
            return carry

        lax.fori_loop(0, PAD_W // 16, padfill, 0)

        def zacc(j, carry):
            acc_v[pl.ds(j * 16, 16)] = zeros16
            return carry

        lax.fori_loop(0, NPAD // 16, zacc, 0)

        def step(j, carry):
            si = sidx_v[pl.ds(j * 16, 16)]
            dv = didx_v[pl.ds(j * 16, 16)]
            vals = plsc.load_gather(p_v, [si])
            plsc.addupdate_scatter(acc_v, [dv], vals)
            return carry

        lax.fori_loop(0, EW // 16, step, 0)
        pltpu.sync_copy(acc_v, out_hbm.at[wid])

    return k(p, srcw, dstw)


def _tc_final(pacc_r, cntc_r, q_r, noise_r, scal):
    def body(sc_ref, pacc_ref, cntc_ref, q_ref, noise_ref, out_ref):
        a = jnp.sum(pacc_ref[...], axis=0)
        h2 = a / cntc_ref[...] + sc_ref[0] + q_ref[...]
        out_ref[...] = (h2 * sc_ref[1] + sc_ref[2]
                        + noise_ref[...] * jnp.exp(h2 * sc_ref[3] + sc_ref[4]))

    R = NPAD // 128
    return pl.pallas_call(
        body,
        in_specs=[
            pl.BlockSpec(memory_space=pltpu.SMEM),
            pl.BlockSpec((NW, R, 128), lambda: (0, 0, 0)),
            pl.BlockSpec((R, 128), lambda: (0, 0)),
            pl.BlockSpec((R, 128), lambda: (0, 0)),
            pl.BlockSpec((R, 128), lambda: (0, 0)),
        ],
        out_specs=pl.BlockSpec((R, 128), lambda: (0, 0)),
        out_shape=jax.ShapeDtypeStruct((R, 128), jnp.float32),
    )(scal, pacc_r, cntc_r, q_r, noise_r)


def kernel(x, edge_index, W1l, b1, W1r, W2l, b2, W2r, Wal, ba, War, wm, bm, wv, bv):
    src = edge_index[0].astype(jnp.int32)
    dst = edge_index[1].astype(jnp.int32)

    part, cntp = _sc_aggregate(x, src, dst)

    xp = jnp.pad(x, ((0, NPAD - N), (0, 0)))
    w2 = jnp.stack([W2l[0], W2r[0]], axis=1)
    xr = _tc_xr(xp, W1r.T, b1.reshape(1, D))
    p2, q2, cntc = _tc_layer(part, cntp, xr, W1l.T, w2)

    pacc = _sc_scalar_aggregate(p2.reshape(NPAD), src, dst)

    R = NPAD // 128
    noise = jax.random.normal(jax.random.key(42), (N, 1), jnp.float32)
    noise_r = jnp.pad(noise[:, 0], (0, NPAD - N)).reshape(R, 128)
    q_r = q2.reshape(R, 128)
    cntc_r = cntc.reshape(R, 128)
    pacc_r = pacc.reshape(NW, R, 128)
    scal = jnp.concatenate([b2, wm.ravel(), bm, wv.ravel(), bv])

    out_r = _tc_final(pacc_r, cntc_r, q_r, noise_r, scal)
    return out_r.reshape(NPAD)[:N][:, None]
